# Initial kernel scaffold; baseline (speedup 1.0000x reference)
import jax
import jax.numpy as jnp
from jax.experimental import pallas as pl


def kernel(inputs, we):
    raise NotImplementedError("write your pallas kernel here")


# speedup vs baseline: 4.3669x; 4.3669x over previous
import functools

import jax
import jax.numpy as jnp
from jax import lax
from jax.experimental import pallas as pl
from jax.experimental.pallas import tpu as pltpu
from jax.experimental.pallas import tpu_sc as plsc

N_VOCAB = 100000
N_CTX = 2048
N_EMBD = 64

B = 1024
L = 200
PAIR = 2

NUM_IDX = B * L * PAIR
NUM_OUT = B * L
IDX_PER_DMA = 128
OUT_PER_DMA = IDX_PER_DMA // 2

_info = plsc.get_sparse_core_info()
NC = _info.num_cores
NS = _info.num_subcores
NW = NC * NS

STEPS = NUM_IDX // (IDX_PER_DMA * NW)
OUT_PER_W = NUM_OUT // NW


def _tec_body(idx_hbm, tab_hbm, out_hbm, idx_v, rows_v, out_v, gsem, osem):
    wid = lax.axis_index("s") * NC + lax.axis_index("c")

    pltpu.sync_copy(idx_hbm.at[wid], idx_v)

    pltpu.async_copy(tab_hbm.at[idx_v.at[0]], rows_v.at[0], gsem)

    def do_step(g, buf):
        @pl.when(g + 1 < STEPS)
        def _():
            pltpu.async_copy(tab_hbm.at[idx_v.at[g + 1]], rows_v.at[1 - buf], gsem)

        pltpu.make_async_copy(tab_hbm.at[idx_v.at[g]], rows_v.at[buf], gsem).wait()

        @pl.when(g >= 2)
        def _():
            pltpu.make_async_copy(
                out_v.at[buf], out_hbm.at[pl.ds(0, OUT_PER_DMA)], osem
            ).wait()

        def row(o, carry):
            for c in range(0, N_EMBD, 16):
                out_v[buf, o, pl.ds(c, 16)] = (
                    rows_v[buf, 2 * o, pl.ds(c, 16)]
                    + rows_v[buf, 2 * o + 1, pl.ds(c, 16)]
                )
            return carry

        lax.fori_loop(0, OUT_PER_DMA, row, 0, unroll=2)

        pltpu.async_copy(
            out_v.at[buf],
            out_hbm.at[pl.ds(wid * OUT_PER_W + g * OUT_PER_DMA, OUT_PER_DMA)],
            osem,
        )

    def outer(i, carry):
        do_step(2 * i, 0)
        do_step(2 * i + 1, 1)
        return carry

    lax.fori_loop(0, STEPS // 2, outer, 0)

    pltpu.make_async_copy(out_v.at[0], out_hbm.at[pl.ds(0, OUT_PER_DMA)], osem).wait()
    pltpu.make_async_copy(out_v.at[1], out_hbm.at[pl.ds(0, OUT_PER_DMA)], osem).wait()


@functools.partial(
    pl.kernel,
    mesh=plsc.VectorSubcoreMesh(core_axis_name="c", subcore_axis_name="s"),
    compiler_params=pltpu.CompilerParams(use_tc_tiling_on_sc=False),
    out_type=jax.ShapeDtypeStruct((NUM_OUT, N_EMBD), jnp.float32),
    scratch_types=[
        pltpu.VMEM((STEPS, IDX_PER_DMA), jnp.int32),
        pltpu.VMEM((2, IDX_PER_DMA, N_EMBD), jnp.float32),
        pltpu.VMEM((2, OUT_PER_DMA, N_EMBD), jnp.float32),
        pltpu.SemaphoreType.DMA,
        pltpu.SemaphoreType.DMA,
    ],
)
def _embed_sum(idx_hbm, tab_hbm, out_hbm, idx_v, rows_v, out_v, gsem, osem):
    _tec_body(idx_hbm, tab_hbm, out_hbm, idx_v, rows_v, out_v, gsem, osem)


@jax.jit
def kernel(inputs, we):
    idx = inputs.reshape(-1).astype(jnp.int32).reshape(NW, STEPS, IDX_PER_DMA)
    out = _embed_sum(idx, we)
    return out.reshape(B, L, N_EMBD)


# speedup vs baseline: 4.4667x; 1.0229x over previous
import functools

import jax
import jax.numpy as jnp
from jax import lax
from jax.experimental import pallas as pl
from jax.experimental.pallas import tpu as pltpu
from jax.experimental.pallas import tpu_sc as plsc

N_VOCAB = 100000
N_CTX = 2048
N_EMBD = 64

B = 1024
L = 200
PAIR = 2

NUM_IDX = B * L * PAIR
NUM_OUT = B * L
IDX_PER_DMA = 100
OUT_PER_DMA = IDX_PER_DMA // 2

_info = plsc.get_sparse_core_info()
NC = _info.num_cores
NS = _info.num_subcores
NW = NC * NS

STEPS = NUM_IDX // (IDX_PER_DMA * NW)
OUT_PER_W = NUM_OUT // NW
B_PER_W = B // NW
STEPS_PER_B = L // OUT_PER_DMA


def _tec_body(idx_hbm, tab_hbm, out_hbm, idx_v, rows_v, out_v, gsem, osem):
    wid = lax.axis_index("s") * NC + lax.axis_index("c")

    pltpu.sync_copy(idx_hbm.at[wid], idx_v)

    pltpu.async_copy(tab_hbm.at[idx_v.at[0]], rows_v.at[0], gsem)

    def do_step(g, buf):
        @pl.when(g + 1 < STEPS)
        def _():
            pltpu.async_copy(tab_hbm.at[idx_v.at[g + 1]], rows_v.at[1 - buf], gsem)

        pltpu.make_async_copy(tab_hbm.at[idx_v.at[g]], rows_v.at[buf], gsem).wait()

        @pl.when(g >= 2)
        def _():
            pltpu.make_async_copy(
                out_v.at[buf], out_hbm.at[0, pl.ds(0, OUT_PER_DMA)], osem
            ).wait()

        def row(o, carry):
            for c in range(0, N_EMBD, 16):
                out_v[buf, o, pl.ds(c, 16)] = (
                    rows_v[buf, 2 * o, pl.ds(c, 16)]
                    + rows_v[buf, 2 * o + 1, pl.ds(c, 16)]
                )
            return carry

        lax.fori_loop(0, OUT_PER_DMA, row, 0, unroll=2)

        pltpu.async_copy(
            out_v.at[buf],
            out_hbm.at[
                wid * B_PER_W + g // STEPS_PER_B,
                pl.ds((g % STEPS_PER_B) * OUT_PER_DMA, OUT_PER_DMA),
            ],
            osem,
        )

    def outer(i, carry):
        do_step(2 * i, 0)
        do_step(2 * i + 1, 1)
        return carry

    lax.fori_loop(0, STEPS // 2, outer, 0)

    pltpu.make_async_copy(out_v.at[0], out_hbm.at[0, pl.ds(0, OUT_PER_DMA)], osem).wait()
    pltpu.make_async_copy(out_v.at[1], out_hbm.at[0, pl.ds(0, OUT_PER_DMA)], osem).wait()


@functools.partial(
    pl.kernel,
    mesh=plsc.VectorSubcoreMesh(core_axis_name="c", subcore_axis_name="s"),
    compiler_params=pltpu.CompilerParams(use_tc_tiling_on_sc=False),
    out_type=jax.ShapeDtypeStruct((B, L, N_EMBD), jnp.float32),
    scratch_types=[
        pltpu.VMEM((STEPS, IDX_PER_DMA), jnp.int32),
        pltpu.VMEM((2, IDX_PER_DMA, N_EMBD), jnp.float32),
        pltpu.VMEM((2, OUT_PER_DMA, N_EMBD), jnp.float32),
        pltpu.SemaphoreType.DMA,
        pltpu.SemaphoreType.DMA,
    ],
)
def _embed_sum(idx_hbm, tab_hbm, out_hbm, idx_v, rows_v, out_v, gsem, osem):
    _tec_body(idx_hbm, tab_hbm, out_hbm, idx_v, rows_v, out_v, gsem, osem)


@jax.jit
def kernel(inputs, we):
    idx = jnp.reshape(inputs.astype(jnp.int32), (NW, STEPS, IDX_PER_DMA))
    return _embed_sum(idx, we)


# speedup vs baseline: 5.9015x; 1.3212x over previous
import functools

import jax
import jax.numpy as jnp
from jax import lax
from jax.experimental import pallas as pl
from jax.experimental.pallas import tpu as pltpu
from jax.experimental.pallas import tpu_sc as plsc

N_EMBD = 64
N_TAB = 102048
B = 1024
L = 200

BLO = 128
BHI = B // BLO

_info = plsc.get_sparse_core_info()
NC = _info.num_cores
NS = _info.num_subcores
NW = NC * NS

LGROUPS = NW // BHI
L_PER_W = L // LGROUPS


def _tec_body(idx_hbm, tab_hbm, out_hbm, idx_v, rows_a, rows_b, out_v, gsem, osem):
    wid = lax.axis_index("s") * NC + lax.axis_index("c")
    j = wid // BHI
    h = wid % BHI

    pltpu.sync_copy(idx_hbm.at[pl.ds(j * L_PER_W, L_PER_W), h], idx_v)

    def fire(k, buf):
        pltpu.async_copy(tab_hbm.at[idx_v.at[k, 0]], rows_a.at[buf], gsem)
        pltpu.async_copy(tab_hbm.at[idx_v.at[k, 1]], rows_b.at[buf], gsem)

    fire(0, 0)

    def do_chunk(k, buf):
        @pl.when(k + 1 < L_PER_W)
        def _():
            fire(k + 1, 1 - buf)

        pltpu.make_async_copy(tab_hbm.at[idx_v.at[k, 0]], rows_a.at[buf], gsem).wait()
        pltpu.make_async_copy(tab_hbm.at[idx_v.at[k, 1]], rows_b.at[buf], gsem).wait()

        @pl.when(k >= 2)
        def _():
            pltpu.make_async_copy(out_v.at[buf], out_hbm.at[0, 0], osem).wait()

        def row(m, carry):
            for c in range(0, N_EMBD, 16):
                out_v[buf, m, pl.ds(c, 16)] = (
                    rows_a[buf, m, pl.ds(c, 16)] + rows_b[buf, m, pl.ds(c, 16)]
                )
            return carry

        lax.fori_loop(0, BLO, row, 0, unroll=2)

        pltpu.async_copy(out_v.at[buf], out_hbm.at[j * L_PER_W + k, h], osem)

    def outer(i, carry):
        do_chunk(2 * i, 0)
        do_chunk(2 * i + 1, 1)
        return carry

    lax.fori_loop(0, L_PER_W // 2, outer, 0)

    pltpu.make_async_copy(out_v.at[0], out_hbm.at[0, 0], osem).wait()
    pltpu.make_async_copy(out_v.at[1], out_hbm.at[0, 0], osem).wait()


@functools.partial(
    pl.kernel,
    mesh=plsc.VectorSubcoreMesh(core_axis_name="c", subcore_axis_name="s"),
    compiler_params=pltpu.CompilerParams(use_tc_tiling_on_sc=False),
    out_type=jax.ShapeDtypeStruct((L, BHI, BLO, N_EMBD), jnp.float32),
    scratch_types=[
        pltpu.VMEM((L_PER_W, 2, BLO), jnp.int32),
        pltpu.VMEM((2, BLO, N_EMBD), jnp.float32),
        pltpu.VMEM((2, BLO, N_EMBD), jnp.float32),
        pltpu.VMEM((2, BLO, N_EMBD), jnp.float32),
        pltpu.SemaphoreType.DMA,
        pltpu.SemaphoreType.DMA,
    ],
)
def _embed_sum(idx_hbm, tab_hbm, out_hbm, idx_v, rows_a, rows_b, out_v, gsem, osem):
    _tec_body(idx_hbm, tab_hbm, out_hbm, idx_v, rows_a, rows_b, out_v, gsem, osem)


@jax.jit
def kernel(inputs, we):
    idx = inputs.astype(jnp.int32).reshape(BHI, BLO, L, 2).transpose(2, 0, 3, 1)
    out = _embed_sum(idx, we)
    return out.transpose(1, 2, 0, 3).reshape(B, L, N_EMBD)
